# baseline (speedup 1.0000x reference)
import jax
import jax.numpy as jnp
from jax.experimental import pallas as pl

EMB = 128
BLK = 24576


def _fill_body(flag_l_ref, flag_r_ref, w_ref, b_ref, out_l_ref, out_r_ref):
    w = w_ref[...]
    b = b_ref[...]
    dn = (((0,), (0,)), ((), ()))
    fl = flag_l_ref[...].reshape(1, BLK)
    fr = flag_r_ref[...].reshape(1, BLK)
    out_l_ref[...] = jax.lax.dot_general(
        fl, w, dn, preferred_element_type=jnp.float32) + b
    out_r_ref[...] = jax.lax.dot_general(
        fr, w, dn, preferred_element_type=jnp.float32) + b


def kernel(x_lig, x_rec, v_lig, v_rec, aa_rec, batch_idx_lig, batch_idx_rec,
           lig_flag, rec_flag, W_ind, b_ind):
    n_lig = lig_flag.shape[0]
    n_rec = rec_flag.shape[0]
    assert n_lig == n_rec
    n = n_lig
    grid = (pl.cdiv(n, BLK),)

    flag_l = lig_flag
    flag_r = rec_flag
    w_row = W_ind.reshape(1, EMB)
    b_row = b_ind.reshape(1, EMB)

    h_lig, h_rec = pl.pallas_call(
        _fill_body,
        grid=grid,
        in_specs=[
            pl.BlockSpec((BLK,), lambda i: (i,)),
            pl.BlockSpec((BLK,), lambda i: (i,)),
            pl.BlockSpec((1, EMB), lambda i: (0, 0)),
            pl.BlockSpec((1, EMB), lambda i: (0, 0)),
        ],
        out_specs=[
            pl.BlockSpec((BLK, EMB), lambda i: (i, 0)),
            pl.BlockSpec((BLK, EMB), lambda i: (i, 0)),
        ],
        out_shape=[
            jax.ShapeDtypeStruct((n, EMB), jnp.float32),
            jax.ShapeDtypeStruct((n, EMB), jnp.float32),
        ],
    )(flag_l, flag_r, w_row, b_row)

    return (x_lig, x_rec, h_lig, h_rec)


# speedup vs baseline: 1.0409x; 1.0409x over previous
import jax
import jax.numpy as jnp
from jax.experimental import pallas as pl

EMB = 128
BLK = 16384


def _fill_body(flag_l_ref, flag_r_ref, w_ref, b_ref, out_l_ref, out_r_ref):
    w = w_ref[...]
    b = b_ref[...]
    dn = (((0,), (0,)), ((), ()))
    fl = flag_l_ref[...].reshape(1, BLK)
    fr = flag_r_ref[...].reshape(1, BLK)
    out_l_ref[...] = jax.lax.dot_general(
        fl, w, dn, preferred_element_type=jnp.float32) + b
    out_r_ref[...] = jax.lax.dot_general(
        fr, w, dn, preferred_element_type=jnp.float32) + b


def kernel(x_lig, x_rec, v_lig, v_rec, aa_rec, batch_idx_lig, batch_idx_rec,
           lig_flag, rec_flag, W_ind, b_ind):
    n_lig = lig_flag.shape[0]
    n_rec = rec_flag.shape[0]
    assert n_lig == n_rec
    n = n_lig
    grid = (pl.cdiv(n, BLK),)

    flag_l = lig_flag
    flag_r = rec_flag
    w_row = W_ind.reshape(1, EMB)
    b_row = b_ind.reshape(1, EMB)

    h_lig, h_rec = pl.pallas_call(
        _fill_body,
        grid=grid,
        in_specs=[
            pl.BlockSpec((BLK,), lambda i: (i,)),
            pl.BlockSpec((BLK,), lambda i: (i,)),
            pl.BlockSpec((1, EMB), lambda i: (0, 0)),
            pl.BlockSpec((1, EMB), lambda i: (0, 0)),
        ],
        out_specs=[
            pl.BlockSpec((BLK, EMB), lambda i: (i, 0)),
            pl.BlockSpec((BLK, EMB), lambda i: (i, 0)),
        ],
        out_shape=[
            jax.ShapeDtypeStruct((n, EMB), jnp.float32),
            jax.ShapeDtypeStruct((n, EMB), jnp.float32),
        ],
    )(flag_l, flag_r, w_row, b_row)

    return (x_lig, x_rec, h_lig, h_rec)
